# baseline (speedup 1.0000x reference)
import functools

import jax
import jax.numpy as jnp
from jax import lax
from jax.experimental import pallas as pl
from jax.experimental.pallas import tpu as pltpu
from jax.experimental.pallas import tpu_sc as plsc

N_LATENT = 32
D = N_LATENT * N_LATENT
BATCH = 16384
H3PACK = 128 // N_LATENT


def _make_decoder():
  info = plsc.get_sparse_core_info()
  NC, NS, L = info.num_cores, info.num_subcores, info.num_lanes
  NW = NC * NS
  RPW = BATCH // NW
  K = 32
  NCH = RPW // K
  NG = K // L

  mesh = plsc.VectorSubcoreMesh(core_axis_name="c", subcore_axis_name="s")

  @functools.partial(
      pl.kernel,
      mesh=mesh,
      compiler_params=pltpu.CompilerParams(needs_layout_passes=False),
      out_type=jax.ShapeDtypeStruct((BATCH * N_LATENT,), jnp.float32),
      scratch_types=[
          pltpu.VMEM((RPW,), jnp.int32),
          pltpu.VMEM((RPW,), jnp.int32),
          pltpu.VMEM((K, D), jnp.float32),
          pltpu.VMEM((K * D,), jnp.float32),
          pltpu.VMEM((K * N_LATENT,), jnp.float32),
          pltpu.VMEM((K, 128), jnp.float32),
          pltpu.VMEM((K * 128,), jnp.float32),
          pltpu.VMEM((K * N_LATENT,), jnp.float32),
          pltpu.SemaphoreType.DMA,
          pltpu.SemaphoreType.DMA,
      ],
  )
  def decoder(u_hbm, sc_hbm, a_hbm, h3_hbm, out_hbm,
              idx_v, idx4_v, rows_v, flat_v, u_v, h3_v, h3f_v, out_v,
              sem_a, sem_h):
    wid = lax.axis_index("s") * NC + lax.axis_index("c")
    base = wid * RPW
    pltpu.sync_copy(sc_hbm.at[pl.ds(base, RPW)], idx_v)
    lanes = lax.iota(jnp.int32, L)

    def div_body(i, c):
      idx4_v[pl.ds(i * L, L)] = lax.shift_right_logical(
          idx_v[pl.ds(i * L, L)], 2)
      return c
    lax.fori_loop(0, RPW // L, div_body, 0)

    def chunk_body(ch, carry):
      row0 = base + ch * K
      cp_a = pltpu.async_copy(
          a_hbm.at[idx_v.at[pl.ds(ch * K, K)]], rows_v, sem_a)
      cp_h = pltpu.async_copy(
          h3_hbm.at[idx4_v.at[pl.ds(ch * K, K)]], h3_v, sem_h)
      pltpu.sync_copy(u_hbm.at[pl.ds(row0 * N_LATENT, K * N_LATENT)], u_v)
      pltpu.sync_copy(u_hbm.at[pl.ds(row0 * N_LATENT, K * N_LATENT)], out_v)
      cp_h.wait()

      def h3copy_body(i, c):
        for j in range(128 // L):
          h3f_v[pl.ds(i * 128 + j * L, L)] = h3_v[i, pl.ds(j * L, L)]
        return c
      lax.fori_loop(0, K, h3copy_body, 0)

      cp_a.wait()

      def copy_body(i, c):
        for j in range(D // L):
          flat_v[pl.ds(i * D + j * L, L)] = rows_v[i, pl.ds(j * L, L)]
        return c
      lax.fori_loop(0, K, copy_body, 0)

      def grp_body(gi, c):
        c_vec = gi * L + lanes
        cb_a = c_vec * D
        cb_u = c_vec * N_LATENT
        iv = idx_v[pl.ds(ch * K + gi * L, L)]
        hb = c_vec * 128 + lax.shift_left(
            lax.bitwise_and(iv, H3PACK - 1), 5)
        u_l = [
            plsc.load_gather(u_v, [cb_u + l]) for l in range(N_LATENT)
        ]

        def g_body(g, cc):
          gbase = cb_a + g * N_LATENT
          acc = plsc.load_gather(h3f_v, [hb + g])
          for l in range(N_LATENT):
            a = plsc.load_gather(flat_v, [gbase + l])
            acc = acc + a * u_l[l]
          plsc.addupdate_scatter(out_v, [cb_u + g], acc)
          return cc
        lax.fori_loop(0, N_LATENT, g_body, 0)
        return c
      lax.fori_loop(0, NG, grp_body, 0)

      pltpu.sync_copy(out_v, out_hbm.at[pl.ds(row0 * N_LATENT, K * N_LATENT)])
      return carry

    lax.fori_loop(0, NCH, chunk_body, 0)

  return decoder


_decoder = _make_decoder()


def kernel(u, sample_covariate, A_s_table, h3_table):
  sc = sample_covariate.astype(jnp.int32)
  h3_packed = h3_table.reshape(-1, 128)
  out = _decoder(u.reshape(-1), sc, A_s_table, h3_packed)
  return out.reshape(BATCH, N_LATENT)


# speedup vs baseline: 1.1060x; 1.1060x over previous
import functools

import jax
import jax.numpy as jnp
from jax import lax
from jax.experimental import pallas as pl
from jax.experimental.pallas import tpu as pltpu
from jax.experimental.pallas import tpu_sc as plsc

N_LATENT = 32
D = N_LATENT * N_LATENT
BATCH = 16384
H3PACK = 128 // N_LATENT


def _make_decoder():
  info = plsc.get_sparse_core_info()
  NC, NS, L = info.num_cores, info.num_subcores, info.num_lanes
  NW = NC * NS
  RPW = BATCH // NW
  K = 32
  NCH = RPW // K
  NG = K // L

  mesh = plsc.VectorSubcoreMesh(core_axis_name="c", subcore_axis_name="s")

  @functools.partial(
      pl.kernel,
      mesh=mesh,
      compiler_params=pltpu.CompilerParams(needs_layout_passes=False),
      out_type=jax.ShapeDtypeStruct((BATCH * N_LATENT,), jnp.float32),
      scratch_types=[
          pltpu.VMEM((RPW,), jnp.int32),
          pltpu.VMEM((RPW,), jnp.int32),
          pltpu.VMEM((K, D), jnp.float32),
          pltpu.VMEM((K * N_LATENT,), jnp.float32),
          pltpu.VMEM((K, 128), jnp.float32),
          pltpu.VMEM((K * N_LATENT,), jnp.float32),
          pltpu.SemaphoreType.DMA,
          pltpu.SemaphoreType.DMA,
      ],
  )
  def decoder(u_hbm, sc_hbm, a_hbm, h3_hbm, out_hbm,
              idx_v, idx4_v, rows_v, u_v, h3_v, out_v, sem_a, sem_h):
    wid = lax.axis_index("s") * NC + lax.axis_index("c")
    base = wid * RPW
    pltpu.sync_copy(sc_hbm.at[pl.ds(base, RPW)], idx_v)
    lanes = lax.iota(jnp.int32, L)

    def div_body(i, c):
      idx4_v[pl.ds(i * L, L)] = lax.shift_right_logical(
          idx_v[pl.ds(i * L, L)], 2)
      return c
    lax.fori_loop(0, RPW // L, div_body, 0)

    def chunk_body(ch, carry):
      row0 = base + ch * K
      cp_a = pltpu.async_copy(
          a_hbm.at[idx_v.at[pl.ds(ch * K, K)]], rows_v, sem_a)
      cp_h = pltpu.async_copy(
          h3_hbm.at[idx4_v.at[pl.ds(ch * K, K)]], h3_v, sem_h)
      pltpu.sync_copy(u_hbm.at[pl.ds(row0 * N_LATENT, K * N_LATENT)], u_v)
      pltpu.sync_copy(u_hbm.at[pl.ds(row0 * N_LATENT, K * N_LATENT)], out_v)
      cp_h.wait()
      cp_a.wait()

      def grp_body(gi, c):
        c_vec = gi * L + lanes
        cb_u = c_vec * N_LATENT
        iv = idx_v[pl.ds(ch * K + gi * L, L)]
        hoff = lax.shift_left(lax.bitwise_and(iv, H3PACK - 1), 5)
        u_l = [
            plsc.load_gather(u_v, [cb_u + l]) for l in range(N_LATENT)
        ]

        def g_body(g, cc):
          col0 = g * N_LATENT
          acc0 = plsc.load_gather(h3_v, [c_vec, hoff + g])
          acc1 = jnp.zeros((L,), jnp.float32)
          acc2 = jnp.zeros((L,), jnp.float32)
          acc3 = jnp.zeros((L,), jnp.float32)
          accs = [acc0, acc1, acc2, acc3]
          for l in range(N_LATENT):
            a = plsc.load_gather(
                rows_v, [c_vec, jnp.full((L,), col0 + l, jnp.int32)])
            accs[l % 4] = accs[l % 4] + a * u_l[l]
          acc = (accs[0] + accs[1]) + (accs[2] + accs[3])
          plsc.addupdate_scatter(out_v, [cb_u + g], acc)
          return cc
        lax.fori_loop(0, N_LATENT, g_body, 0)
        return c
      lax.fori_loop(0, NG, grp_body, 0)

      pltpu.sync_copy(out_v, out_hbm.at[pl.ds(row0 * N_LATENT, K * N_LATENT)])
      return carry

    lax.fori_loop(0, NCH, chunk_body, 0)

  return decoder


_decoder = _make_decoder()


def kernel(u, sample_covariate, A_s_table, h3_table):
  sc = sample_covariate.astype(jnp.int32)
  h3_packed = h3_table.reshape(-1, 128)
  out = _decoder(u.reshape(-1), sc, A_s_table, h3_packed)
  return out.reshape(BATCH, N_LATENT)


# speedup vs baseline: 1.3130x; 1.1871x over previous
import functools

import jax
import jax.numpy as jnp
from jax import lax
from jax.experimental import pallas as pl
from jax.experimental.pallas import tpu as pltpu
from jax.experimental.pallas import tpu_sc as plsc

N_LATENT = 32
D = N_LATENT * N_LATENT
BATCH = 16384
H3PACK = 128 // N_LATENT


def _make_decoder():
  info = plsc.get_sparse_core_info()
  NC, NS, L = info.num_cores, info.num_subcores, info.num_lanes
  NW = NC * NS
  RPW = BATCH // NW
  K = 32
  NCH = RPW // K
  NG = K // L

  mesh = plsc.VectorSubcoreMesh(core_axis_name="c", subcore_axis_name="s")

  @functools.partial(
      pl.kernel,
      mesh=mesh,
      compiler_params=pltpu.CompilerParams(needs_layout_passes=False),
      out_type=jax.ShapeDtypeStruct((BATCH * N_LATENT,), jnp.float32),
      scratch_types=[
          pltpu.VMEM((RPW,), jnp.int32),
          pltpu.VMEM((RPW,), jnp.int32),
          pltpu.VMEM((K, D), jnp.float32),
          pltpu.VMEM((K * N_LATENT,), jnp.float32),
          pltpu.VMEM((K, 128), jnp.float32),
          pltpu.VMEM((K * N_LATENT,), jnp.float32),
          pltpu.SemaphoreType.DMA,
          pltpu.SemaphoreType.DMA,
      ],
  )
  def decoder(u_hbm, sc_hbm, a_hbm, h3_hbm, out_hbm,
              idx_v, idx4_v, rows_v, u_v, h3_v, out_v, sem_a, sem_h):
    wid = lax.axis_index("s") * NC + lax.axis_index("c")
    base = wid * RPW
    pltpu.sync_copy(sc_hbm.at[pl.ds(base, RPW)], idx_v)
    lanes = lax.iota(jnp.int32, L)

    def div_body(i, c):
      idx4_v[pl.ds(i * L, L)] = lax.shift_right_logical(
          idx_v[pl.ds(i * L, L)], 2)
      return c
    lax.fori_loop(0, RPW // L, div_body, 0)

    def chunk_body(ch, carry):
      row0 = base + ch * K
      cp_a = pltpu.async_copy(
          a_hbm.at[idx_v.at[pl.ds(ch * K, K)]], rows_v, sem_a)
      cp_h = pltpu.async_copy(
          h3_hbm.at[idx4_v.at[pl.ds(ch * K, K)]], h3_v, sem_h)
      pltpu.sync_copy(u_hbm.at[pl.ds(row0 * N_LATENT, K * N_LATENT)], u_v)
      cp_h.wait()
      cp_a.wait()

      def grp_body(gi, c):
        c_vec = gi * L + lanes
        cb_u = c_vec * N_LATENT
        iv = idx_v[pl.ds(ch * K + gi * L, L)]
        hoff = lax.shift_left(lax.bitwise_and(iv, H3PACK - 1), 5)
        u_l = [
            plsc.load_gather(u_v, [cb_u + l]) for l in range(N_LATENT)
        ]
        for g in range(N_LATENT):
          col0 = g * N_LATENT
          h3g = plsc.load_gather(h3_v, [c_vec, hoff + g])
          accs = [h3g + u_l[g], None, None, None]
          for half in range(2):
            a_h = [
                plsc.load_gather(
                    rows_v,
                    [c_vec, jnp.full((L,), col0 + half * L + l, jnp.int32)])
                for l in range(L)
            ]
            for l in range(L):
              k = l % 4
              term = a_h[l] * u_l[half * L + l]
              accs[k] = term if accs[k] is None else accs[k] + term
          acc = (accs[0] + accs[1]) + (accs[2] + accs[3])
          plsc.store_scatter(out_v, [cb_u + g], acc)
        return c
      lax.fori_loop(0, NG, grp_body, 0)

      pltpu.sync_copy(out_v, out_hbm.at[pl.ds(row0 * N_LATENT, K * N_LATENT)])
      return carry

    lax.fori_loop(0, NCH, chunk_body, 0)

  return decoder


_decoder = _make_decoder()


def kernel(u, sample_covariate, A_s_table, h3_table):
  sc = sample_covariate.astype(jnp.int32)
  h3_packed = h3_table.reshape(-1, 128)
  out = _decoder(u.reshape(-1), sc, A_s_table, h3_packed)
  return out.reshape(BATCH, N_LATENT)


# speedup vs baseline: 2.3564x; 1.7947x over previous
import functools

import jax
import jax.numpy as jnp
import numpy as np
from jax import lax
from jax.experimental import pallas as pl
from jax.experimental.pallas import tpu as pltpu
from jax.experimental.pallas import tpu_sc as plsc

N_LATENT = 32
D = N_LATENT * N_LATENT
BATCH = 16384
H3PACK = 128 // N_LATENT


def _make_gather():
  info = plsc.get_sparse_core_info()
  NC, NS, L = info.num_cores, info.num_subcores, info.num_lanes
  NW = NC * NS
  RPW = BATCH // NW
  K = 32
  NCH = RPW // K

  mesh = plsc.VectorSubcoreMesh(core_axis_name="c", subcore_axis_name="s")

  @functools.partial(
      pl.kernel,
      mesh=mesh,
      compiler_params=pltpu.CompilerParams(needs_layout_passes=False),
      out_type=(
          jax.ShapeDtypeStruct((BATCH, D), jnp.float32),
          jax.ShapeDtypeStruct((BATCH, 128), jnp.float32),
      ),
      scratch_types=[
          pltpu.VMEM((RPW,), jnp.int32),
          pltpu.VMEM((RPW,), jnp.int32),
          pltpu.VMEM((K, D), jnp.float32),
          pltpu.VMEM((K, D), jnp.float32),
          pltpu.VMEM((K, 128), jnp.float32),
          pltpu.VMEM((K, 128), jnp.float32),
          pltpu.SemaphoreType.DMA,
          pltpu.SemaphoreType.DMA,
          pltpu.SemaphoreType.DMA,
          pltpu.SemaphoreType.DMA,
      ],
  )
  def gather(sc_hbm, a_hbm, h3_hbm, aout_hbm, h3out_hbm,
             idx_v, idx4_v, rowsA0, rowsA1, h3q0, h3q1,
             semA0, semA1, semH0, semH1):
    wid = lax.axis_index("s") * NC + lax.axis_index("c")
    base = wid * RPW
    pltpu.sync_copy(sc_hbm.at[pl.ds(base, RPW)], idx_v)

    def div_body(i, c):
      idx4_v[pl.ds(i * L, L)] = lax.shift_right_logical(
          idx_v[pl.ds(i * L, L)], 2)
      return c
    lax.fori_loop(0, RPW // L, div_body, 0)

    rowsA = (rowsA0, rowsA1)
    h3q = (h3q0, h3q1)
    semA = (semA0, semA1)
    semH = (semH0, semH1)

    def issue(ch, b):
      cpa = pltpu.async_copy(
          a_hbm.at[idx_v.at[pl.ds(ch * K, K)]], rowsA[b], semA[b])
      cph = pltpu.async_copy(
          h3_hbm.at[idx4_v.at[pl.ds(ch * K, K)]], h3q[b], semH[b])
      return cpa, cph

    def process(ch, b, cpa, cph):
      row0 = base + ch * K
      cph.wait()
      pltpu.sync_copy(h3q[b], h3out_hbm.at[pl.ds(row0, K)])
      cpa.wait()
      pltpu.sync_copy(rowsA[b], aout_hbm.at[pl.ds(row0, K)])

    cpa_cur, cph_cur = issue(0, 0)
    for p in range(NCH // 2):
      cpa1, cph1 = issue(2 * p + 1, 1)
      process(2 * p, 0, cpa_cur, cph_cur)
      if 2 * p + 2 < NCH:
        cpa_cur, cph_cur = issue(2 * p + 2, 0)
      process(2 * p + 1, 1, cpa1, cph1)

  return gather


_gather = _make_gather()


def _tc_body(a_ref, u_ref, h3q_ref, sc_ref, s_ref, o_ref):
  a = a_ref[...]
  u = u_ref[...]
  u_rep = jnp.tile(u, (1, N_LATENT))
  h2 = jnp.dot(a * u_rep, s_ref[...], preferred_element_type=jnp.float32)
  q = lax.bitwise_and(sc_ref[...], H3PACK - 1)
  h3 = jnp.zeros_like(u)
  for j in range(H3PACK):
    sel = (q == j).astype(jnp.float32)
    h3 = h3 + sel * h3q_ref[:, j * N_LATENT:(j + 1) * N_LATENT]
  o_ref[...] = u + h2 + h3


_TC_BLK = 256


@jax.jit
def _decode(u, sc, sc2d, a_table, h3_packed, s_mat):
  a_g, h3q_g = _gather(sc, a_table, h3_packed)
  grid = BATCH // _TC_BLK
  return pl.pallas_call(
      _tc_body,
      grid=(grid,),
      in_specs=[
          pl.BlockSpec((_TC_BLK, D), lambda i: (i, 0)),
          pl.BlockSpec((_TC_BLK, N_LATENT), lambda i: (i, 0)),
          pl.BlockSpec((_TC_BLK, 128), lambda i: (i, 0)),
          pl.BlockSpec((_TC_BLK, 1), lambda i: (i, 0)),
          pl.BlockSpec((D, N_LATENT), lambda i: (0, 0)),
      ],
      out_specs=pl.BlockSpec((_TC_BLK, N_LATENT), lambda i: (i, 0)),
      out_shape=jax.ShapeDtypeStruct((BATCH, N_LATENT), jnp.float32),
  )(a_g, u, h3q_g, sc2d, s_mat)


_S_MAT = np.zeros((D, N_LATENT), np.float32)
_S_MAT[np.arange(D), np.arange(D) // N_LATENT] = 1.0


def kernel(u, sample_covariate, A_s_table, h3_table):
  sc = sample_covariate.astype(jnp.int32)
  h3_packed = h3_table.reshape(-1, 128)
  return _decode(u, sc, sc.reshape(BATCH, 1), A_s_table, h3_packed,
                 jnp.asarray(_S_MAT))


# speedup vs baseline: 2.3980x; 1.0176x over previous
import functools

import jax
import jax.numpy as jnp
import numpy as np
from jax import lax
from jax.experimental import pallas as pl
from jax.experimental.pallas import tpu as pltpu
from jax.experimental.pallas import tpu_sc as plsc

N_LATENT = 32
D = N_LATENT * N_LATENT
BATCH = 16384
H3PACK = 128 // N_LATENT


def _make_gather():
  info = plsc.get_sparse_core_info()
  NC, NS, L = info.num_cores, info.num_subcores, info.num_lanes
  NW = NC * NS
  RPW = BATCH // NW
  K = 32
  NCH = RPW // K

  mesh = plsc.VectorSubcoreMesh(core_axis_name="c", subcore_axis_name="s")

  @functools.partial(
      pl.kernel,
      mesh=mesh,
      compiler_params=pltpu.CompilerParams(needs_layout_passes=False),
      out_type=(
          jax.ShapeDtypeStruct((BATCH, D), jnp.float32),
          jax.ShapeDtypeStruct((BATCH, 128), jnp.float32),
      ),
      scratch_types=[
          pltpu.VMEM((RPW,), jnp.int32),
          pltpu.VMEM((RPW,), jnp.int32),
          pltpu.VMEM((K, D), jnp.float32),
          pltpu.VMEM((K, D), jnp.float32),
          pltpu.VMEM((K, 128), jnp.float32),
          pltpu.VMEM((K, 128), jnp.float32),
          pltpu.SemaphoreType.DMA,
          pltpu.SemaphoreType.DMA,
          pltpu.SemaphoreType.DMA,
          pltpu.SemaphoreType.DMA,
      ],
  )
  def gather(sc_hbm, a_hbm, h3_hbm, aout_hbm, h3out_hbm,
             idx_v, idx4_v, rowsA0, rowsA1, h3q0, h3q1,
             semA0, semA1, semH0, semH1):
    wid = lax.axis_index("s") * NC + lax.axis_index("c")
    base = wid * RPW
    pltpu.sync_copy(sc_hbm.at[pl.ds(base, RPW)], idx_v)

    def div_body(i, c):
      idx4_v[pl.ds(i * L, L)] = lax.shift_right_logical(
          idx_v[pl.ds(i * L, L)], 2)
      return c
    lax.fori_loop(0, RPW // L, div_body, 0)

    rowsA = (rowsA0, rowsA1)
    h3q = (h3q0, h3q1)
    semA = (semA0, semA1)
    semH = (semH0, semH1)

    def issue(ch, b):
      cpa = pltpu.async_copy(
          a_hbm.at[idx_v.at[pl.ds(ch * K, K)]], rowsA[b], semA[b])
      cph = pltpu.async_copy(
          h3_hbm.at[idx4_v.at[pl.ds(ch * K, K)]], h3q[b], semH[b])
      return cpa, cph

    def process(ch, b, cpa, cph):
      row0 = base + ch * K
      cph.wait()
      pltpu.sync_copy(h3q[b], h3out_hbm.at[pl.ds(row0, K)])
      cpa.wait()
      pltpu.sync_copy(rowsA[b], aout_hbm.at[pl.ds(row0, K)])

    cpa_cur, cph_cur = issue(0, 0)
    for p in range(NCH // 2):
      cpa1, cph1 = issue(2 * p + 1, 1)
      process(2 * p, 0, cpa_cur, cph_cur)
      if 2 * p + 2 < NCH:
        cpa_cur, cph_cur = issue(2 * p + 2, 0)
      process(2 * p + 1, 1, cpa1, cph1)

  return gather


_gather = _make_gather()


def _tc_body(a_ref, u_ref, h3q_ref, sc_ref, s_ref, t_ref, o_ref):
  a = a_ref[...]
  u = u_ref[...]
  u_rep = jnp.dot(u, t_ref[...], preferred_element_type=jnp.float32)
  h2 = jnp.dot(a * u_rep, s_ref[...], preferred_element_type=jnp.float32)
  q = lax.bitwise_and(sc_ref[...], H3PACK - 1)
  h3 = jnp.zeros_like(u)
  for j in range(H3PACK):
    sel = (q == j).astype(jnp.float32)
    h3 = h3 + sel * h3q_ref[:, j * N_LATENT:(j + 1) * N_LATENT]
  o_ref[...] = u + h2 + h3


_TC_BLK = 256


@jax.jit
def _decode(u, sc, sc2d, a_table, h3_packed, s_mat, t_mat):
  a_g, h3q_g = _gather(sc, a_table, h3_packed)
  grid = BATCH // _TC_BLK
  return pl.pallas_call(
      _tc_body,
      grid=(grid,),
      in_specs=[
          pl.BlockSpec((_TC_BLK, D), lambda i: (i, 0)),
          pl.BlockSpec((_TC_BLK, N_LATENT), lambda i: (i, 0)),
          pl.BlockSpec((_TC_BLK, 128), lambda i: (i, 0)),
          pl.BlockSpec((_TC_BLK, 1), lambda i: (i, 0)),
          pl.BlockSpec((D, N_LATENT), lambda i: (0, 0)),
          pl.BlockSpec((N_LATENT, D), lambda i: (0, 0)),
      ],
      out_specs=pl.BlockSpec((_TC_BLK, N_LATENT), lambda i: (i, 0)),
      out_shape=jax.ShapeDtypeStruct((BATCH, N_LATENT), jnp.float32),
  )(a_g, u, h3q_g, sc2d, s_mat, t_mat)


_S_MAT = np.zeros((D, N_LATENT), np.float32)
_S_MAT[np.arange(D), np.arange(D) // N_LATENT] = 1.0
_T_MAT = np.zeros((N_LATENT, D), np.float32)
_T_MAT[np.arange(D) % N_LATENT, np.arange(D)] = 1.0


def kernel(u, sample_covariate, A_s_table, h3_table):
  sc = sample_covariate.astype(jnp.int32)
  h3_packed = h3_table.reshape(-1, 128)
  return _decode(u, sc, sc.reshape(BATCH, 1), A_s_table, h3_packed,
                 jnp.asarray(_S_MAT), jnp.asarray(_T_MAT))
